# baseline (speedup 1.0000x reference)
import functools

import jax
import jax.numpy as jnp
from jax import lax
from jax.experimental import pallas as pl
from jax.experimental.pallas import tpu as pltpu
from jax.experimental.pallas import tpu_sc as plsc

B = 16384
D = 64
NC = 2
NS = 16
NW = NC * NS
B_PER_W = B // NW
CH = 128
NCH = B_PER_W // CH

_sc_mesh = plsc.VectorSubcoreMesh(core_axis_name="c", subcore_axis_name="s")


@functools.partial(
    pl.kernel,
    mesh=_sc_mesh,
    out_type=[
        jax.ShapeDtypeStruct((B, D), jnp.float32),
        jax.ShapeDtypeStruct((B, D), jnp.float32),
    ],
    scratch_types=[
        pltpu.VMEM((NCH, CH), jnp.int32),
        pltpu.VMEM((NCH, CH), jnp.int32),
        pltpu.VMEM((B_PER_W, D), jnp.float32),
        pltpu.VMEM((B_PER_W, D), jnp.float32),
        pltpu.SemaphoreType.DMA,
    ],
    compiler_params=pltpu.CompilerParams(use_tc_tiling_on_sc=False),
)
def _sc_gather(user_hbm, item_hbm, su_hbm, ti_hbm, u_out, i_out,
               uidx_v, iidx_v, urows_v, irows_v, sem):
    wid = lax.axis_index("s") * NC + lax.axis_index("c")
    base = wid * B_PER_W
    pltpu.sync_copy(user_hbm.at[pl.ds(wid * NCH, NCH)], uidx_v)
    pltpu.sync_copy(item_hbm.at[pl.ds(wid * NCH, NCH)], iidx_v)
    handles = []
    for j in range(NCH):
        handles.append(pltpu.async_copy(
            su_hbm.at[uidx_v.at[j]], urows_v.at[pl.ds(j * CH, CH)], sem))
        handles.append(pltpu.async_copy(
            ti_hbm.at[iidx_v.at[j]], irows_v.at[pl.ds(j * CH, CH)], sem))
    for h in handles:
        h.wait()
    pltpu.sync_copy(urows_v, u_out.at[pl.ds(base, B_PER_W)])
    pltpu.sync_copy(irows_v, i_out.at[pl.ds(base, B_PER_W)])


BLK = 2048


def _mlp_body(u_ref, i_ref, w1_ref, b1_ref, w2_ref, b2_ref, w3_ref, b3_ref,
              w4_ref, b4_ref, out_ref):
    u = u_ref[...]
    h = jnp.maximum(jnp.dot(u, w1_ref[...], preferred_element_type=jnp.float32)
                    + b1_ref[...], 0.0)
    h = jnp.maximum(jnp.dot(h, w2_ref[...], preferred_element_type=jnp.float32)
                    + b2_ref[...], 0.0)
    h = jnp.maximum(jnp.dot(h, w3_ref[...], preferred_element_type=jnp.float32)
                    + b3_ref[...], 0.0)
    fu = jnp.dot(h, w4_ref[...], preferred_element_type=jnp.float32) + b4_ref[...]
    out_ref[0, 0, :] = jnp.sum(fu * i_ref[...], axis=1)


def _tc_mlp(u_rows, i_rows, w1t, b1, w2t, b2, w3t, b3, w4t, b4):
    grid = B // BLK
    return pl.pallas_call(
        _mlp_body,
        grid=(grid,),
        in_specs=[
            pl.BlockSpec((BLK, D), lambda i: (i, 0)),
            pl.BlockSpec((BLK, D), lambda i: (i, 0)),
            pl.BlockSpec(w1t.shape, lambda i: (0, 0)),
            pl.BlockSpec(b1.shape, lambda i: (0, 0)),
            pl.BlockSpec(w2t.shape, lambda i: (0, 0)),
            pl.BlockSpec(b2.shape, lambda i: (0, 0)),
            pl.BlockSpec(w3t.shape, lambda i: (0, 0)),
            pl.BlockSpec(b3.shape, lambda i: (0, 0)),
            pl.BlockSpec(w4t.shape, lambda i: (0, 0)),
            pl.BlockSpec(b4.shape, lambda i: (0, 0)),
        ],
        out_specs=pl.BlockSpec((1, 1, BLK), lambda i: (i, 0, 0)),
        out_shape=jax.ShapeDtypeStruct((grid, 1, BLK), jnp.float32),
    )(u_rows, i_rows, w1t, b1, w2t, b2, w3t, b3, w4t, b4)


def kernel(user, item, su_emb, ti_emb, W1, b1, W2, b2, W3, b3, W4, b4):
    user2d = user.astype(jnp.int32).reshape(B // CH, CH)
    item2d = item.astype(jnp.int32).reshape(B // CH, CH)
    u_rows, i_rows = _sc_gather(user2d, item2d, su_emb, ti_emb)
    score = _tc_mlp(u_rows, i_rows,
                    W1.T, b1.reshape(1, -1),
                    W2.T, b2.reshape(1, -1),
                    W3.T, b3.reshape(1, -1),
                    W4.T, b4.reshape(1, -1))
    return score.reshape(B)


# speedup vs baseline: 1.9070x; 1.9070x over previous
import functools

import jax
import jax.numpy as jnp
from jax import lax
from jax.experimental import pallas as pl
from jax.experimental.pallas import tpu as pltpu
from jax.experimental.pallas import tpu_sc as plsc

B = 16384
D = 64
DP = 128
NC = 2
NS = 16
NW = NC * NS
L = 16
N_USER = 1000000
N_TITEM = 100000
UCOLS = (N_USER + DP - 1) // DP
ICOLS = (N_TITEM + DP - 1) // DP
MAXM = B + L
NOUT = B + L

_sc_mesh = plsc.VectorSubcoreMesh(core_axis_name="c", subcore_axis_name="s")


def _splat(x):
    return jnp.full((L,), x, jnp.int32)


@functools.partial(
    pl.kernel,
    mesh=_sc_mesh,
    out_type=[
        jax.ShapeDtypeStruct((NOUT, DP), jnp.float32),
        jax.ShapeDtypeStruct((NOUT, DP), jnp.float32),
    ],
    scratch_types=[
        pltpu.VMEM((B // DP, DP), jnp.int32),
        pltpu.VMEM((MAXM,), jnp.int32),
        pltpu.VMEM((MAXM,), jnp.int32),
        pltpu.VMEM((272,), jnp.int32),
        pltpu.VMEM((272,), jnp.int32),
        pltpu.VMEM((2, D, DP), jnp.float32),
        pltpu.VMEM((DP, DP), jnp.float32),
        pltpu.VMEM((1, DP), jnp.int32),
        pltpu.VMEM((32,), jnp.int32),
        pltpu.VMEM((32,), jnp.int32),
        pltpu.SemaphoreType.DMA,
        pltpu.SemaphoreType.DMA,
        pltpu.SemaphoreType.DMA,
    ],
    compiler_params=pltpu.CompilerParams(use_tc_tiling_on_sc=True,
                                         needs_layout_passes=False),
)
def _sc_gather(user_hbm, item_hbm, su_hbm, ti_hbm, u_out, i_out,
               idx_v, mu_v, mb_v, pres_v, cols_v, slab_v, stage_v, bidx_v,
               lt_v, bt_v, sem0, sem1, sems):
    wid = lax.axis_index("s") * NC + lax.axis_index("c")
    iota = lax.iota(jnp.int32, L)
    zero16 = jnp.zeros((L,), jnp.int32)

    def one_table(src_idx_hbm, tab_hbm, out_hbm, ncols_total, fill0):
        lo = (wid * ncols_total) // NW
        hi = ((wid + 1) * ncols_total) // NW
        pltpu.sync_copy(src_idx_hbm, idx_v)
        for j in range(272 // L):
            pres_v[pl.ds(j * L, L)] = zero16

        def scan_row(r, cnt):
            for k in range(DP // L):
                u = plsc.load_gather(idx_v, [_splat(r), k * L + iota])
                c = jax.lax.shift_right_logical(u, 7)
                m = (c >= lo) & (c < hi)
                plsc.store_compressed(mu_v.at[pl.ds(cnt, L)], u, mask=m)
                bvec = r * DP + k * L + iota
                plsc.store_compressed(mb_v.at[pl.ds(cnt, L)], bvec, mask=m)
                plsc.store_scatter(pres_v, [c - lo],
                                   jnp.ones((L,), jnp.int32), mask=m)
                cnt = cnt + plsc.all_reduce_population_count(m)[0]
            return cnt
        cnt = lax.fori_loop(0, B // DP, scan_row, jnp.int32(0))

        def col_chunk(j, ncol):
            p = pres_v[pl.ds(j * L, L)]
            m = p > 0
            plsc.store_compressed(cols_v.at[pl.ds(ncol, L)], lo + j * L + iota, mask=m)
            return ncol + plsc.all_reduce_population_count(m)[0]
        ncol = lax.fori_loop(0, 272 // L, col_chunk, jnp.int32(0))

        def slab_dma(ci, buf):
            col = cols_v[pl.ds(ci, L)][0]
            off = pl.multiple_of(col * DP, DP)
            return pltpu.async_copy(
                tab_hbm.at[:, pl.ds(off, DP)], slab_v.at[buf], sem0)

        nchunks = (cnt + L - 1) // L

        @pl.when(ncol > 0)
        def _():
            slab_dma(0, 0)

        def per_col(ci, fill):
            pltpu.make_async_copy(
                tab_hbm.at[:, pl.ds(0, DP)], slab_v.at[ci % 2], sem0).wait()

            @pl.when(ci + 1 < ncol)
            def _():
                slab_dma(ci + 1, (ci + 1) % 2)
            col = cols_v[pl.ds(ci, L)][0]
            buf = ci % 2

            def per_chunk(j, fill):
                u = mu_v[pl.ds(j * L, L)]
                bvec = mb_v[pl.ds(j * L, L)]
                c = jax.lax.shift_right_logical(u, 7)
                m = (c == col) & ((j * L + iota) < cnt)
                mcnt = plsc.all_reduce_population_count(m)[0]

                @pl.when(mcnt > 0)
                def _():
                    plsc.store_compressed(lt_v.at[pl.ds(0, L)], u & 127, mask=m)
                    plsc.store_compressed(bt_v.at[pl.ds(0, L)], bvec, mask=m)

                def per_match(t, fill):
                    lane = plsc.load_gather(lt_v, [_splat(t)])
                    b = plsc.load_gather(bt_v, [_splat(t)])
                    slot = fill & (DP - 1)
                    for k in range(D // L):
                        g = plsc.load_gather(slab_v.at[buf],
                                             [k * L + iota, lane])
                        plsc.store_scatter(stage_v, [_splat(slot), k * L + iota], g)
                    plsc.store_scatter(bidx_v, [zero16, _splat(slot)], b,
                                       mask=(iota == 0))
                    fill = fill + 1

                    @pl.when((fill & (DP - 1)) == 0)
                    def _():
                        pltpu.async_copy(
                            stage_v, out_hbm.at[bidx_v.at[0]], sems).wait()
                    return fill
                return lax.fori_loop(0, mcnt, per_match, fill)
            return lax.fori_loop(0, nchunks, per_chunk, fill)

        fill = lax.fori_loop(0, ncol, per_col, fill0)

        rem = fill & (DP - 1)

        @pl.when(rem > 0)
        def _():
            def pad_slot(s, _):
                plsc.store_scatter(bidx_v, [zero16, _splat(s)],
                                   _splat(B) + (_splat(s) & 15), mask=(iota == 0))
                return _
            lax.fori_loop(rem, DP, pad_slot, jnp.int32(0))
            pltpu.async_copy(stage_v, out_hbm.at[bidx_v.at[0]], sems).wait()

    one_table(user_hbm, su_hbm, u_out, UCOLS, jnp.int32(0))
    one_table(item_hbm, ti_hbm, i_out, ICOLS, jnp.int32(0))


BLK = 2048


def _mlp_body(u_ref, i_ref, w1_ref, b1_ref, w2_ref, b2_ref,
              w3_ref, b3_ref, w4_ref, b4_ref, out_ref):
    u = u_ref[:, :D]
    iv = i_ref[:, :D]
    h = jnp.maximum(jnp.dot(u, w1_ref[...], preferred_element_type=jnp.float32)
                    + b1_ref[...], 0.0)
    h = jnp.maximum(jnp.dot(h, w2_ref[...], preferred_element_type=jnp.float32)
                    + b2_ref[...], 0.0)
    h = jnp.maximum(jnp.dot(h, w3_ref[...], preferred_element_type=jnp.float32)
                    + b3_ref[...], 0.0)
    fu = jnp.dot(h, w4_ref[...], preferred_element_type=jnp.float32) + b4_ref[...]
    out_ref[0, 0, :] = jnp.sum(fu * iv, axis=1)


def _tc_mlp(u_rows, i_rows, w1t, b1, w2t, b2, w3t, b3, w4t, b4):
    grid = B // BLK
    return pl.pallas_call(
        _mlp_body,
        grid=(grid,),
        in_specs=[
            pl.BlockSpec((BLK, DP), lambda i: (i, 0)),
            pl.BlockSpec((BLK, DP), lambda i: (i, 0)),
            pl.BlockSpec(w1t.shape, lambda i: (0, 0)),
            pl.BlockSpec(b1.shape, lambda i: (0, 0)),
            pl.BlockSpec(w2t.shape, lambda i: (0, 0)),
            pl.BlockSpec(b2.shape, lambda i: (0, 0)),
            pl.BlockSpec(w3t.shape, lambda i: (0, 0)),
            pl.BlockSpec(b3.shape, lambda i: (0, 0)),
            pl.BlockSpec(w4t.shape, lambda i: (0, 0)),
            pl.BlockSpec(b4.shape, lambda i: (0, 0)),
        ],
        out_specs=pl.BlockSpec((1, 1, BLK), lambda i: (i, 0, 0)),
        out_shape=jax.ShapeDtypeStruct((grid, 1, BLK), jnp.float32),
    )(u_rows, i_rows, w1t, b1, w2t, b2, w3t, b3, w4t, b4)


def kernel(user, item, su_emb, ti_emb, W1, b1, W2, b2, W3, b3, W4, b4):
    user = user.astype(jnp.int32)
    item = item.astype(jnp.int32)
    u_rows, i_rows = _sc_gather(user.reshape(B // DP, DP),
                                item.reshape(B // DP, DP),
                                su_emb.T, ti_emb.T)
    score = _tc_mlp(u_rows[:B], i_rows[:B],
                    W1.T, b1.reshape(1, -1),
                    W2.T, b2.reshape(1, -1),
                    W3.T, b3.reshape(1, -1),
                    W4.T, b4.reshape(1, -1))
    return score.reshape(B)
